# baseline (speedup 1.0000x reference)
import functools

import jax
import jax.numpy as jnp
from jax import lax
from jax.experimental import pallas as pl
from jax.experimental.pallas import tpu as pltpu
from jax.experimental.pallas import tpu_sc as plsc

CHUNK = 128


@functools.lru_cache(maxsize=None)
def _build(B, S, V, D):
    info = plsc.get_sparse_core_info()
    NC, NS, L = info.num_cores, info.num_subcores, info.num_lanes
    NW = NC * NS
    NB = B // CHUNK
    n_chunks = S * NB
    chunks_per_w = n_chunks // NW
    mesh = plsc.VectorSubcoreMesh(core_axis_name="c", subcore_axis_name="s")

    @functools.partial(
        pl.kernel,
        mesh=mesh,
        compiler_params=pltpu.CompilerParams(
            use_tc_tiling_on_sc=False, needs_layout_passes=False
        ),
        out_type=jax.ShapeDtypeStruct((S, D // 8, NB, 8, CHUNK), jnp.float32),
        scratch_types=[
            pltpu.VMEM((chunks_per_w, CHUNK), jnp.int32),
            pltpu.VMEM((CHUNK, D), jnp.float32),
            pltpu.VMEM((D // 8, 8, CHUNK), jnp.float32),
            pltpu.SemaphoreType.DMA,
        ],
    )
    def k(idx_hbm, table_hbm, out_hbm, idx_v, rows_v, tbuf_v, sem):
        wid = lax.axis_index("s") * NC + lax.axis_index("c")
        g0 = wid * chunks_per_w
        pltpu.sync_copy(idx_hbm.at[pl.ds(g0, chunks_per_w)], idx_v)

        lane = lax.iota(jnp.int32, L)
        row_ids = [lane + jnp.int32(gi * L) for gi in range(CHUNK // L)]

        def body(gg, carry):
            g = g0 + gg
            s = g // NB
            bb = g % NB
            pltpu.async_copy(table_hbm.at[idx_v.at[gg]], rows_v, sem).wait()

            def col(c, carry2):
                cvec = jnp.full((L,), 0, jnp.int32) + c
                for gi in range(CHUNK // L):
                    vals = plsc.load_gather(rows_v, [row_ids[gi], cvec])
                    tbuf_v[c // 8, c % 8, pl.ds(gi * L, L)] = vals
                return carry2

            lax.fori_loop(0, D, col, 0)
            pltpu.sync_copy(tbuf_v, out_hbm.at[s, :, bb])
            return carry

        lax.fori_loop(0, chunks_per_w, body, 0)

    return k


def kernel(x, table):
    B, S = x.shape
    V, D = table.shape
    idx2d = x.T.reshape(S * B // CHUNK, CHUNK).astype(jnp.int32)
    out = _build(B, S, V, D)(idx2d, table)
    return out.transpose(2, 4, 0, 1, 3).reshape(B, S, D)
